# baseline (speedup 1.0000x reference)
import functools

import jax
import jax.numpy as jnp
from jax import lax
from jax.experimental import pallas as pl
from jax.experimental.pallas import tpu as pltpu
from jax.experimental.pallas import tpu_sc as plsc

N = 512
DIN = 512
DIM = 128
E = 16384
BI = 8

_SC_CORES = 2
_SC_SUBCORES = 16
_NW = _SC_CORES * _SC_SUBCORES
_ROWS_PER_W = N // _NW
_LANES = 16


def _count_sc_body(src_hbm, dst_hbm, out_hbm, src_v, dst_v, c_v):
    wid = lax.axis_index("c") * _SC_SUBCORES + lax.axis_index("s")
    lo = wid * _ROWS_PER_W
    pltpu.sync_copy(src_hbm, src_v)
    pltpu.sync_copy(dst_hbm, dst_v)
    zeros = jnp.zeros((_LANES,), jnp.float32)

    def _zero(j, _):
        c_v[pl.ds(j * _LANES, _LANES)] = zeros
        return 0

    lax.fori_loop(0, _ROWS_PER_W * N // _LANES, _zero, 0)

    def _accum(i, _):
        d = dst_v[pl.ds(i * _LANES, _LANES)]
        s = src_v[pl.ds(i * _LANES, _LANES)]
        m = (d >= lo) & (d < lo + _ROWS_PER_W)
        idx = (d - lo) * N + s
        cnt, last = plsc.scan_count(idx, m)
        plsc.addupdate_scatter(c_v, [idx], cnt.astype(jnp.float32),
                               mask=last)
        return 0

    lax.fori_loop(0, E // _LANES, _accum, 0)
    pltpu.sync_copy(c_v, out_hbm.at[pl.ds(lo * N, _ROWS_PER_W * N)])


_count_sc = functools.partial(
    pl.kernel,
    out_type=jax.ShapeDtypeStruct((N * N,), jnp.float32),
    mesh=plsc.VectorSubcoreMesh(core_axis_name="c", subcore_axis_name="s"),
    compiler_params=pltpu.CompilerParams(needs_layout_passes=False),
    scratch_types=[
        pltpu.VMEM((E,), jnp.int32),
        pltpu.VMEM((E,), jnp.int32),
        pltpu.VMEM((_ROWS_PER_W * N,), jnp.float32),
    ],
)(_count_sc_body)


def _ln(t, g, b, eps=1e-5):
    mu = jnp.mean(t, axis=-1, keepdims=True)
    d = t - mu
    var = jnp.mean(d * d, axis=-1, keepdims=True)
    return d * jax.lax.rsqrt(var + eps) * g + b


def _gelu(t):
    return 0.5 * t * (1.0 + jax.lax.erf(t * 0.7071067811865476))


def _mlp_kernel(c_ref, x_ref,
                w1a_ref, b1a_ref, g1a_ref, be1a_ref, w2a_ref, b2a_ref,
                w1b_ref, b1b_ref, g1b_ref, be1b_ref, w2b_ref, b2b_ref,
                ew1_ref, eb1_ref,
                a_ref, b_out_ref):
    C = c_ref[...]
    x = x_ref[...]

    def gin(h, w1, b1, g1, be1, w2, b2):
        t = jnp.dot(h, w1, preferred_element_type=jnp.float32) + b1
        t = _gelu(_ln(t, g1, be1))
        return jnp.dot(t, w2, preferred_element_type=jnp.float32) + b2

    agg1 = jnp.dot(C, x, preferred_element_type=jnp.float32)
    h = gin(x + agg1, w1a_ref[...], b1a_ref[...], g1a_ref[...],
            be1a_ref[...], w2a_ref[...], b2a_ref[...])
    agg2 = jnp.dot(C, h, preferred_element_type=jnp.float32)
    h = gin(h + agg2, w1b_ref[...], b1b_ref[...], g1b_ref[...],
            be1b_ref[...], w2b_ref[...], b2b_ref[...])
    ew1 = ew1_ref[...]
    a_ref[...] = (jnp.dot(h, ew1[:DIM, :], preferred_element_type=jnp.float32)
                  + eb1_ref[...])
    b_out_ref[...] = jnp.dot(h, ew1[DIM:, :], preferred_element_type=jnp.float32)


def _pair_kernel(a_ref, b_ref, g_ref, be_ref, w2_ref, b2_ref, o_ref):
    A = a_ref[...]
    B = b_ref[...]
    z = A[:, None, :] + B[None, :, :]
    mu = jnp.mean(z, axis=-1, keepdims=True)
    d = z - mu
    var = jnp.mean(d * d, axis=-1, keepdims=True)
    y = d * jax.lax.rsqrt(var + 1e-5) * g_ref[...][None] + be_ref[...][None]
    y = _gelu(y)
    o = jnp.sum(y * w2_ref[...][None], axis=-1) + b2_ref[...]
    o_ref[...] = jax.nn.sigmoid(o)


def kernel(x, edge_index, c1_W1, c1_b1, c1_g1, c1_be1, c1_W2, c1_b2,
           c2_W1, c2_b1, c2_g1, c2_be1, c2_W2, c2_b2,
           e_W1, e_b1, e_g1, e_be1, e_W2, e_b2):
    ei = edge_index.astype(jnp.int32)
    src = ei[0]
    dst = ei[1]

    C = _count_sc(src, dst).reshape(N, N)

    r = lambda v: v.reshape(1, -1)
    A, B = pl.pallas_call(
        _mlp_kernel,
        out_shape=(jax.ShapeDtypeStruct((N, DIM), jnp.float32),
                   jax.ShapeDtypeStruct((N, DIM), jnp.float32)),
    )(C, x,
      c1_W1, r(c1_b1), r(c1_g1), r(c1_be1), c1_W2, r(c1_b2),
      c2_W1, r(c2_b1), r(c2_g1), r(c2_be1), c2_W2, r(c2_b2),
      e_W1, r(e_b1))

    out = pl.pallas_call(
        _pair_kernel,
        grid=(N // BI,),
        in_specs=[pl.BlockSpec((BI, DIM), lambda i: (i, 0)),
                  pl.BlockSpec((N, DIM), lambda i: (0, 0)),
                  pl.BlockSpec((1, DIM), lambda i: (0, 0)),
                  pl.BlockSpec((1, DIM), lambda i: (0, 0)),
                  pl.BlockSpec((1, DIM), lambda i: (0, 0)),
                  pl.BlockSpec((1, 1), lambda i: (0, 0))],
        out_specs=pl.BlockSpec((BI, N), lambda i: (i, 0)),
        out_shape=jax.ShapeDtypeStruct((N, N), jnp.float32),
    )(A, B, r(e_g1), r(e_be1), e_W2.reshape(1, DIM), e_b2.reshape(1, 1))
    return out


# speedup vs baseline: 1.2882x; 1.2882x over previous
import functools

import jax
import jax.numpy as jnp
from jax import lax
from jax.experimental import pallas as pl
from jax.experimental.pallas import tpu as pltpu
from jax.experimental.pallas import tpu_sc as plsc

N = 512
DIN = 512
DIM = 128
E = 16384
BI = 8

_SC_CORES = 2
_SC_SUBCORES = 16
_NW = _SC_CORES * _SC_SUBCORES
_ROWS_PER_W = N // _NW
_LANES = 16


def _count_sc_body(src_hbm, dst_hbm, out_hbm, src_v, dst_v, c_v):
    wid = lax.axis_index("c") * _SC_SUBCORES + lax.axis_index("s")
    lo = wid * _ROWS_PER_W
    pltpu.sync_copy(src_hbm, src_v)
    pltpu.sync_copy(dst_hbm, dst_v)
    zeros = jnp.zeros((_LANES,), jnp.float32)

    def _zero(j, _):
        c_v[pl.ds(j * _LANES, _LANES)] = zeros
        return 0

    lax.fori_loop(0, _ROWS_PER_W * N // _LANES, _zero, 0)

    def _accum(i, _):
        d = dst_v[pl.ds(i * _LANES, _LANES)]
        s = src_v[pl.ds(i * _LANES, _LANES)]
        m = (d >= lo) & (d < lo + _ROWS_PER_W)
        idx = (d - lo) * N + s
        cnt, last = plsc.scan_count(idx, m)
        plsc.addupdate_scatter(c_v, [idx], cnt.astype(jnp.float32),
                               mask=last)
        return 0

    lax.fori_loop(0, E // _LANES, _accum, 0)
    pltpu.sync_copy(c_v, out_hbm.at[pl.ds(lo * N, _ROWS_PER_W * N)])


_count_sc = functools.partial(
    pl.kernel,
    out_type=jax.ShapeDtypeStruct((N * N,), jnp.float32),
    mesh=plsc.VectorSubcoreMesh(core_axis_name="c", subcore_axis_name="s"),
    compiler_params=pltpu.CompilerParams(needs_layout_passes=False),
    scratch_types=[
        pltpu.VMEM((E,), jnp.int32),
        pltpu.VMEM((E,), jnp.int32),
        pltpu.VMEM((_ROWS_PER_W * N,), jnp.float32),
    ],
)(_count_sc_body)


def _ln(t, g, b, eps=1e-5):
    mu = jnp.mean(t, axis=-1, keepdims=True)
    d = t - mu
    var = jnp.mean(d * d, axis=-1, keepdims=True)
    return d * jax.lax.rsqrt(var + eps) * g + b


def _gelu(t):
    return 0.5 * t * (1.0 + jax.lax.erf(t * 0.7071067811865476))


def _mlp_kernel(c_ref, x_ref,
                w1a_ref, b1a_ref, g1a_ref, be1a_ref, w2a_ref, b2a_ref,
                w1b_ref, b1b_ref, g1b_ref, be1b_ref, w2b_ref, b2b_ref,
                ew1_ref, eb1_ref, g_ref,
                ac_ref, bc_ref, acg_ref, bcg_ref, va_ref, vb_ref):
    C = c_ref[...]
    x = x_ref[...]

    def gin(h, w1, b1, g1, be1, w2, b2):
        t = jnp.dot(h, w1, preferred_element_type=jnp.float32) + b1
        t = _gelu(_ln(t, g1, be1))
        return jnp.dot(t, w2, preferred_element_type=jnp.float32) + b2

    agg1 = jnp.dot(C, x, preferred_element_type=jnp.float32)
    h = gin(x + agg1, w1a_ref[...], b1a_ref[...], g1a_ref[...],
            be1a_ref[...], w2a_ref[...], b2a_ref[...])
    agg2 = jnp.dot(C, h, preferred_element_type=jnp.float32)
    h = gin(h + agg2, w1b_ref[...], b1b_ref[...], g1b_ref[...],
            be1b_ref[...], w2b_ref[...], b2b_ref[...])
    ew1 = ew1_ref[...]
    g = g_ref[...]
    A = (jnp.dot(h, ew1[:DIM, :], preferred_element_type=jnp.float32)
         + eb1_ref[...])
    B = jnp.dot(h, ew1[DIM:, :], preferred_element_type=jnp.float32)
    Ac = A - jnp.mean(A, axis=-1, keepdims=True)
    Bc = B - jnp.mean(B, axis=-1, keepdims=True)
    ac_ref[...] = Ac
    bc_ref[...] = Bc
    acg_ref[...] = Ac * g
    bcg_ref[...] = Bc * g
    va_ref[...] = jnp.mean(Ac * Ac, axis=-1, keepdims=True)
    ones_row = jnp.ones((1, DIM), jnp.float32)
    vb_ref[...] = lax.dot_general(
        ones_row, Bc * Bc, (((1,), (1,)), ((), ())),
        preferred_element_type=jnp.float32) * (1.0 / DIM)


def _pair_kernel(ac_ref, bc_ref, acg_ref, bcg_ref, va_ref, vb_ref,
                 be_ref, w2_ref, b2_ref, o_ref):
    Ac = ac_ref[...]
    Bc = bc_ref[...]
    P = lax.dot_general(Ac, Bc, (((1,), (1,)), ((), ())),
                        preferred_element_type=jnp.float32)
    var = va_ref[...] + vb_ref[...] + (2.0 / DIM) * P
    rstd = jax.lax.rsqrt(var + 1e-5)
    t = ((acg_ref[...][:, None, :] + bcg_ref[...][None, :, :])
         * rstd[:, :, None] + be_ref[...][None, :, :])
    y = _gelu(t)
    o = jnp.sum(y * w2_ref[...][None, :, :], axis=-1)
    o_ref[...] = jax.nn.sigmoid(o + b2_ref[...])


def kernel(x, edge_index, c1_W1, c1_b1, c1_g1, c1_be1, c1_W2, c1_b2,
           c2_W1, c2_b1, c2_g1, c2_be1, c2_W2, c2_b2,
           e_W1, e_b1, e_g1, e_be1, e_W2, e_b2):
    ei = edge_index.astype(jnp.int32)
    src = ei[0]
    dst = ei[1]

    C = _count_sc(src, dst).reshape(N, N)

    r = lambda v: v.reshape(1, -1)
    Ac, Bc, Acg, Bcg, Va, Vb = pl.pallas_call(
        _mlp_kernel,
        out_shape=(jax.ShapeDtypeStruct((N, DIM), jnp.float32),
                   jax.ShapeDtypeStruct((N, DIM), jnp.float32),
                   jax.ShapeDtypeStruct((N, DIM), jnp.float32),
                   jax.ShapeDtypeStruct((N, DIM), jnp.float32),
                   jax.ShapeDtypeStruct((N, 1), jnp.float32),
                   jax.ShapeDtypeStruct((1, N), jnp.float32)),
    )(C, x,
      c1_W1, r(c1_b1), r(c1_g1), r(c1_be1), c1_W2, r(c1_b2),
      c2_W1, r(c2_b1), r(c2_g1), r(c2_be1), c2_W2, r(c2_b2),
      e_W1, r(e_b1), r(e_g1))

    out = pl.pallas_call(
        _pair_kernel,
        grid=(N // BI,),
        in_specs=[pl.BlockSpec((BI, DIM), lambda i: (i, 0)),
                  pl.BlockSpec((N, DIM), lambda i: (0, 0)),
                  pl.BlockSpec((BI, DIM), lambda i: (i, 0)),
                  pl.BlockSpec((N, DIM), lambda i: (0, 0)),
                  pl.BlockSpec((BI, 1), lambda i: (i, 0)),
                  pl.BlockSpec((1, N), lambda i: (0, 0)),
                  pl.BlockSpec((1, DIM), lambda i: (0, 0)),
                  pl.BlockSpec((1, DIM), lambda i: (0, 0)),
                  pl.BlockSpec((1, 1), lambda i: (0, 0))],
        out_specs=pl.BlockSpec((BI, N), lambda i: (i, 0)),
        out_shape=jax.ShapeDtypeStruct((N, N), jnp.float32),
    )(Ac, Bc, Acg, Bcg, Va, Vb, r(e_be1), e_W2.reshape(1, DIM),
      e_b2.reshape(1, 1))
    return out


# speedup vs baseline: 1.6023x; 1.2438x over previous
import functools

import jax
import jax.numpy as jnp
from jax import lax
from jax.experimental import pallas as pl
from jax.experimental.pallas import tpu as pltpu
from jax.experimental.pallas import tpu_sc as plsc

N = 512
DIN = 512
DIM = 128
E = 16384
BI = 8

_SC_CORES = 2
_SC_SUBCORES = 16
_NW = _SC_CORES * _SC_SUBCORES
_ROWS_PER_W = N // _NW
_LANES = 16


def _count_sc_body(src_hbm, dst_hbm, out_hbm, src_v, dst_v, c_v):
    wid = lax.axis_index("c") * _SC_SUBCORES + lax.axis_index("s")
    lo = wid * _ROWS_PER_W
    pltpu.sync_copy(src_hbm, src_v)
    pltpu.sync_copy(dst_hbm, dst_v)
    zeros = jnp.zeros((_LANES,), jnp.float32)

    def _zero(j, _):
        c_v[pl.ds(j * _LANES, _LANES)] = zeros
        return 0

    lax.fori_loop(0, _ROWS_PER_W * N // _LANES, _zero, 0)

    def _accum(i, _):
        d = dst_v[pl.ds(i * _LANES, _LANES)]
        s = src_v[pl.ds(i * _LANES, _LANES)]
        m = (d >= lo) & (d < lo + _ROWS_PER_W)
        idx = (d - lo) * N + s
        cnt, last = plsc.scan_count(idx, m)
        plsc.addupdate_scatter(c_v, [idx], cnt.astype(jnp.float32),
                               mask=last)
        return 0

    lax.fori_loop(0, E // _LANES, _accum, 0)
    pltpu.sync_copy(c_v, out_hbm.at[pl.ds(lo * N, _ROWS_PER_W * N)])


_count_sc = functools.partial(
    pl.kernel,
    out_type=jax.ShapeDtypeStruct((N * N,), jnp.float32),
    mesh=plsc.VectorSubcoreMesh(core_axis_name="c", subcore_axis_name="s"),
    compiler_params=pltpu.CompilerParams(needs_layout_passes=False),
    scratch_types=[
        pltpu.VMEM((E,), jnp.int32),
        pltpu.VMEM((E,), jnp.int32),
        pltpu.VMEM((_ROWS_PER_W * N,), jnp.float32),
    ],
)(_count_sc_body)


def _ln(t, g, b, eps=1e-5):
    mu = jnp.mean(t, axis=-1, keepdims=True)
    d = t - mu
    var = jnp.mean(d * d, axis=-1, keepdims=True)
    return d * jax.lax.rsqrt(var + eps) * g + b


def _gelu(t):
    return 0.5 * t * (1.0 + jax.lax.erf(t * 0.7071067811865476))


def _mlp_kernel(c_ref, x_ref,
                w1a_ref, b1a_ref, g1a_ref, be1a_ref, w2a_ref, b2a_ref,
                w1b_ref, b1b_ref, g1b_ref, be1b_ref, w2b_ref, b2b_ref,
                ew1_ref, eb1_ref, g_ref,
                ac_ref, bc_ref, acg_ref, bcg_ref, va_ref, vb_ref):
    C = c_ref[...]
    x = x_ref[...]

    def gin(h, w1, b1, g1, be1, w2, b2):
        t = jnp.dot(h, w1, preferred_element_type=jnp.float32) + b1
        t = _gelu(_ln(t, g1, be1))
        return jnp.dot(t, w2, preferred_element_type=jnp.float32) + b2

    agg1 = jnp.dot(C, x, preferred_element_type=jnp.float32)
    h = gin(x + agg1, w1a_ref[...], b1a_ref[...], g1a_ref[...],
            be1a_ref[...], w2a_ref[...], b2a_ref[...])
    agg2 = jnp.dot(C, h, preferred_element_type=jnp.float32)
    h = gin(h + agg2, w1b_ref[...], b1b_ref[...], g1b_ref[...],
            be1b_ref[...], w2b_ref[...], b2b_ref[...])
    ew1 = ew1_ref[...]
    g = g_ref[...]
    A = (jnp.dot(h, ew1[:DIM, :], preferred_element_type=jnp.float32)
         + eb1_ref[...])
    B = jnp.dot(h, ew1[DIM:, :], preferred_element_type=jnp.float32)
    Ac = A - jnp.mean(A, axis=-1, keepdims=True)
    Bc = B - jnp.mean(B, axis=-1, keepdims=True)
    ac_ref[...] = Ac
    bc_ref[...] = Bc
    acg_ref[...] = Ac * g
    bcg_ref[...] = Bc * g
    va_ref[...] = jnp.mean(Ac * Ac, axis=-1, keepdims=True)
    ones_row = jnp.ones((1, DIM), jnp.float32)
    vb_ref[...] = lax.dot_general(
        ones_row, Bc * Bc, (((1,), (1,)), ((), ())),
        preferred_element_type=jnp.float32) * (1.0 / DIM)


def _pair_kernel(ac_ref, bc_ref, acg_ref, bcgt_ref, va_ref, vb_ref,
                 ben_ref, w2_ref, b2_ref, o_ref):
    Ac = ac_ref[...]
    Bc = bc_ref[...]
    P = lax.dot_general(Ac, Bc, (((1,), (1,)), ((), ())),
                        preferred_element_type=jnp.float32)
    var = va_ref[...] + vb_ref[...] + (2.0 / DIM) * P
    rstd = jax.lax.rsqrt(var + 1e-5)
    t = ((acg_ref[...][:, :, None] + bcgt_ref[...][None, :, :])
         * rstd[:, None, :] + ben_ref[...][None, :, :])
    y = _gelu(t)
    w2 = w2_ref[...]
    rows = [lax.dot_general(w2, y[k], (((1,), (0,)), ((), ())),
                            preferred_element_type=jnp.float32)
            for k in range(BI)]
    o = jnp.concatenate(rows, axis=0)
    o_ref[...] = jax.nn.sigmoid(o + b2_ref[...])


def kernel(x, edge_index, c1_W1, c1_b1, c1_g1, c1_be1, c1_W2, c1_b2,
           c2_W1, c2_b1, c2_g1, c2_be1, c2_W2, c2_b2,
           e_W1, e_b1, e_g1, e_be1, e_W2, e_b2):
    ei = edge_index.astype(jnp.int32)
    src = ei[0]
    dst = ei[1]

    C = _count_sc(src, dst).reshape(N, N)

    r = lambda v: v.reshape(1, -1)
    Ac, Bc, Acg, Bcg, Va, Vb = pl.pallas_call(
        _mlp_kernel,
        out_shape=(jax.ShapeDtypeStruct((N, DIM), jnp.float32),
                   jax.ShapeDtypeStruct((N, DIM), jnp.float32),
                   jax.ShapeDtypeStruct((N, DIM), jnp.float32),
                   jax.ShapeDtypeStruct((N, DIM), jnp.float32),
                   jax.ShapeDtypeStruct((N, 1), jnp.float32),
                   jax.ShapeDtypeStruct((1, N), jnp.float32)),
    )(C, x,
      c1_W1, r(c1_b1), r(c1_g1), r(c1_be1), c1_W2, r(c1_b2),
      c2_W1, r(c2_b1), r(c2_g1), r(c2_be1), c2_W2, r(c2_b2),
      e_W1, r(e_b1), r(e_g1))

    BcgT = Bcg.T
    beN = jnp.broadcast_to(e_be1.reshape(DIM, 1), (DIM, N))
    out = pl.pallas_call(
        _pair_kernel,
        grid=(N // BI,),
        in_specs=[pl.BlockSpec((BI, DIM), lambda i: (i, 0)),
                  pl.BlockSpec((N, DIM), lambda i: (0, 0)),
                  pl.BlockSpec((BI, DIM), lambda i: (i, 0)),
                  pl.BlockSpec((DIM, N), lambda i: (0, 0)),
                  pl.BlockSpec((BI, 1), lambda i: (i, 0)),
                  pl.BlockSpec((1, N), lambda i: (0, 0)),
                  pl.BlockSpec((DIM, N), lambda i: (0, 0)),
                  pl.BlockSpec((1, DIM), lambda i: (0, 0)),
                  pl.BlockSpec((1, 1), lambda i: (0, 0))],
        out_specs=pl.BlockSpec((BI, N), lambda i: (i, 0)),
        out_shape=jax.ShapeDtypeStruct((N, N), jnp.float32),
    )(Ac, Bc, Acg, BcgT, Va, Vb, beN, e_W2.reshape(1, DIM),
      e_b2.reshape(1, 1))
    return out


# speedup vs baseline: 1.9281x; 1.2034x over previous
import functools

import jax
import jax.numpy as jnp
from jax import lax
from jax.experimental import pallas as pl
from jax.experimental.pallas import tpu as pltpu
from jax.experimental.pallas import tpu_sc as plsc

N = 512
DIN = 512
DIM = 128
E = 16384
BI = 8

_SC_CORES = 2
_SC_SUBCORES = 16
_ROWS_PER_W = N // _SC_SUBCORES
_HE = E // _SC_CORES
_LANES = 16


def _count_sc_body(src_hbm, dst_hbm, out_hbm, src_v, dst_v, c_v):
    cid = lax.axis_index("c")
    sid = lax.axis_index("s")
    lo = sid * _ROWS_PER_W
    pltpu.sync_copy(src_hbm.at[pl.ds(cid * _HE, _HE)], src_v)
    pltpu.sync_copy(dst_hbm.at[pl.ds(cid * _HE, _HE)], dst_v)
    zeros = jnp.zeros((_LANES,), jnp.float32)

    @plsc.parallel_loop(0, _ROWS_PER_W * N // _LANES, unroll=8)
    def _zero(j):
        c_v[pl.ds(j * _LANES, _LANES)] = zeros

    @plsc.parallel_loop(0, _HE // _LANES, unroll=4)
    def _accum(i):
        d = dst_v[pl.ds(i * _LANES, _LANES)]
        s = src_v[pl.ds(i * _LANES, _LANES)]
        m = (d >= lo) & (d < lo + _ROWS_PER_W)
        idx = (d - lo) * N + s
        cnt, last = plsc.scan_count(idx, m)
        plsc.addupdate_scatter(c_v, [idx], cnt.astype(jnp.float32),
                               mask=last)

    pltpu.sync_copy(
        c_v, out_hbm.at[pl.ds((cid * N + lo) * N, _ROWS_PER_W * N)])


_count_sc = functools.partial(
    pl.kernel,
    out_type=jax.ShapeDtypeStruct((_SC_CORES * N * N,), jnp.float32),
    mesh=plsc.VectorSubcoreMesh(core_axis_name="c", subcore_axis_name="s"),
    compiler_params=pltpu.CompilerParams(needs_layout_passes=False),
    scratch_types=[
        pltpu.VMEM((_HE,), jnp.int32),
        pltpu.VMEM((_HE,), jnp.int32),
        pltpu.VMEM((_ROWS_PER_W * N,), jnp.float32),
    ],
)(_count_sc_body)


def _ln(t, g, b, eps=1e-5):
    mu = jnp.mean(t, axis=-1, keepdims=True)
    d = t - mu
    var = jnp.mean(d * d, axis=-1, keepdims=True)
    return d * jax.lax.rsqrt(var + eps) * g + b


def _gelu(t):
    return 0.5 * t * (1.0 + jax.lax.erf(t * 0.7071067811865476))


def _mlp_kernel(c_ref, x_ref,
                w1a_ref, b1a_ref, g1a_ref, be1a_ref, w2a_ref, b2a_ref,
                w1b_ref, b1b_ref, g1b_ref, be1b_ref, w2b_ref, b2b_ref,
                ew1_ref, eb1_ref, g_ref,
                ac_ref, bc_ref, acg_ref, bcg_ref, va_ref, vb_ref):
    C = c_ref[0] + c_ref[1]
    x = x_ref[...]

    def gin(h, w1, b1, g1, be1, w2, b2):
        t = jnp.dot(h, w1, preferred_element_type=jnp.float32) + b1
        t = _gelu(_ln(t, g1, be1))
        return jnp.dot(t, w2, preferred_element_type=jnp.float32) + b2

    agg1 = jnp.dot(C, x, preferred_element_type=jnp.float32)
    h = gin(x + agg1, w1a_ref[...], b1a_ref[...], g1a_ref[...],
            be1a_ref[...], w2a_ref[...], b2a_ref[...])
    agg2 = jnp.dot(C, h, preferred_element_type=jnp.float32)
    h = gin(h + agg2, w1b_ref[...], b1b_ref[...], g1b_ref[...],
            be1b_ref[...], w2b_ref[...], b2b_ref[...])
    ew1 = ew1_ref[...]
    g = g_ref[...]
    A = (jnp.dot(h, ew1[:DIM, :], preferred_element_type=jnp.float32)
         + eb1_ref[...])
    B = jnp.dot(h, ew1[DIM:, :], preferred_element_type=jnp.float32)
    Ac = A - jnp.mean(A, axis=-1, keepdims=True)
    Bc = B - jnp.mean(B, axis=-1, keepdims=True)
    ac_ref[...] = Ac
    bc_ref[...] = Bc
    acg_ref[...] = Ac * g
    bcg_ref[...] = Bc * g
    va_ref[...] = jnp.mean(Ac * Ac, axis=-1, keepdims=True)
    ones_row = jnp.ones((1, DIM), jnp.float32)
    vb_ref[...] = lax.dot_general(
        ones_row, Bc * Bc, (((1,), (1,)), ((), ())),
        preferred_element_type=jnp.float32) * (1.0 / DIM)


def _pair_kernel(ac_ref, bc_ref, acg_ref, bcgt_ref, va_ref, vb_ref,
                 ben_ref, w2_ref, b2_ref, o_ref):
    Ac = ac_ref[...]
    Bc = bc_ref[...]
    P = lax.dot_general(Ac, Bc, (((1,), (1,)), ((), ())),
                        preferred_element_type=jnp.float32)
    var = va_ref[...] + vb_ref[...] + (2.0 / DIM) * P
    rstd = jax.lax.rsqrt(var + 1e-5)
    t = ((acg_ref[...][:, :, None] + bcgt_ref[...][None, :, :])
         * rstd[:, None, :] + ben_ref[...][None, :, :])
    y = _gelu(t)
    w2 = w2_ref[...]
    rows = [lax.dot_general(w2, y[k], (((1,), (0,)), ((), ())),
                            preferred_element_type=jnp.float32)
            for k in range(BI)]
    o = jnp.concatenate(rows, axis=0)
    o_ref[...] = jax.nn.sigmoid(o + b2_ref[...])


def kernel(x, edge_index, c1_W1, c1_b1, c1_g1, c1_be1, c1_W2, c1_b2,
           c2_W1, c2_b1, c2_g1, c2_be1, c2_W2, c2_b2,
           e_W1, e_b1, e_g1, e_be1, e_W2, e_b2):
    ei = edge_index.astype(jnp.int32)
    src = ei[0]
    dst = ei[1]

    C = _count_sc(src, dst).reshape(_SC_CORES, N, N)

    r = lambda v: v.reshape(1, -1)
    Ac, Bc, Acg, Bcg, Va, Vb = pl.pallas_call(
        _mlp_kernel,
        out_shape=(jax.ShapeDtypeStruct((N, DIM), jnp.float32),
                   jax.ShapeDtypeStruct((N, DIM), jnp.float32),
                   jax.ShapeDtypeStruct((N, DIM), jnp.float32),
                   jax.ShapeDtypeStruct((N, DIM), jnp.float32),
                   jax.ShapeDtypeStruct((N, 1), jnp.float32),
                   jax.ShapeDtypeStruct((1, N), jnp.float32)),
    )(C, x,
      c1_W1, r(c1_b1), r(c1_g1), r(c1_be1), c1_W2, r(c1_b2),
      c2_W1, r(c2_b1), r(c2_g1), r(c2_be1), c2_W2, r(c2_b2),
      e_W1, r(e_b1), r(e_g1))

    BcgT = Bcg.T
    beN = jnp.broadcast_to(e_be1.reshape(DIM, 1), (DIM, N))
    out = pl.pallas_call(
        _pair_kernel,
        grid=(N // BI,),
        in_specs=[pl.BlockSpec((BI, DIM), lambda i: (i, 0)),
                  pl.BlockSpec((N, DIM), lambda i: (0, 0)),
                  pl.BlockSpec((BI, DIM), lambda i: (i, 0)),
                  pl.BlockSpec((DIM, N), lambda i: (0, 0)),
                  pl.BlockSpec((BI, 1), lambda i: (i, 0)),
                  pl.BlockSpec((1, N), lambda i: (0, 0)),
                  pl.BlockSpec((DIM, N), lambda i: (0, 0)),
                  pl.BlockSpec((1, DIM), lambda i: (0, 0)),
                  pl.BlockSpec((1, 1), lambda i: (0, 0))],
        out_specs=pl.BlockSpec((BI, N), lambda i: (i, 0)),
        out_shape=jax.ShapeDtypeStruct((N, N), jnp.float32),
    )(Ac, Bc, Acg, BcgT, Va, Vb, beN, e_W2.reshape(1, DIM),
      e_b2.reshape(1, 1))
    return out


# speedup vs baseline: 2.1546x; 1.1175x over previous
import functools

import jax
import jax.numpy as jnp
from jax import lax
from jax.experimental import pallas as pl
from jax.experimental.pallas import tpu as pltpu
from jax.experimental.pallas import tpu_sc as plsc

N = 512
DIN = 512
DIM = 128
E = 16384
BI = 16

_SC_CORES = 2
_SC_SUBCORES = 16
_ROWS_PER_W = N // _SC_SUBCORES
_HE = E // _SC_CORES
_LANES = 16


def _count_sc_body(src_hbm, dst_hbm, out_hbm, src_v, dst_v, c_v):
    cid = lax.axis_index("c")
    sid = lax.axis_index("s")
    lo = sid * _ROWS_PER_W
    pltpu.sync_copy(src_hbm.at[pl.ds(cid * _HE, _HE)], src_v)
    pltpu.sync_copy(dst_hbm.at[pl.ds(cid * _HE, _HE)], dst_v)
    zeros = jnp.zeros((_LANES,), jnp.float32)

    @plsc.parallel_loop(0, _ROWS_PER_W * N // _LANES, unroll=8)
    def _zero(j):
        c_v[pl.ds(j * _LANES, _LANES)] = zeros

    @plsc.parallel_loop(0, _HE // _LANES, unroll=4)
    def _accum(i):
        d = dst_v[pl.ds(i * _LANES, _LANES)]
        s = src_v[pl.ds(i * _LANES, _LANES)]
        m = (d >= lo) & (d < lo + _ROWS_PER_W)
        idx = (d - lo) * N + s
        cnt, last = plsc.scan_count(idx, m)
        plsc.addupdate_scatter(c_v, [idx], cnt.astype(jnp.float32),
                               mask=last)

    pltpu.sync_copy(
        c_v, out_hbm.at[pl.ds((cid * N + lo) * N, _ROWS_PER_W * N)])


_count_sc = functools.partial(
    pl.kernel,
    out_type=jax.ShapeDtypeStruct((_SC_CORES * N * N,), jnp.float32),
    mesh=plsc.VectorSubcoreMesh(core_axis_name="c", subcore_axis_name="s"),
    compiler_params=pltpu.CompilerParams(needs_layout_passes=False),
    scratch_types=[
        pltpu.VMEM((_HE,), jnp.int32),
        pltpu.VMEM((_HE,), jnp.int32),
        pltpu.VMEM((_ROWS_PER_W * N,), jnp.float32),
    ],
)(_count_sc_body)


def _ln(t, g, b, eps=1e-5):
    mu = jnp.mean(t, axis=-1, keepdims=True)
    d = t - mu
    var = jnp.mean(d * d, axis=-1, keepdims=True)
    return d * jax.lax.rsqrt(var + eps) * g + b


def _gelu(t):
    return 0.5 * t * (1.0 + jax.lax.erf(t * 0.7071067811865476))


def _mlp_kernel(c_ref, x_ref,
                w1a_ref, b1a_ref, g1a_ref, be1a_ref, w2a_ref, b2a_ref,
                w1b_ref, b1b_ref, g1b_ref, be1b_ref, w2b_ref, b2b_ref,
                ew1_ref, eb1_ref, g_ref,
                ac_ref, bc_ref, acg_ref, bcg_ref, va_ref, vb_ref):
    C = c_ref[0] + c_ref[1]
    x = x_ref[...]

    def gin(h, w1, b1, g1, be1, w2, b2):
        t = jnp.dot(h, w1, preferred_element_type=jnp.float32) + b1
        t = _gelu(_ln(t, g1, be1))
        return jnp.dot(t, w2, preferred_element_type=jnp.float32) + b2

    agg1 = jnp.dot(C, x, preferred_element_type=jnp.float32)
    h = gin(x + agg1, w1a_ref[...], b1a_ref[...], g1a_ref[...],
            be1a_ref[...], w2a_ref[...], b2a_ref[...])
    agg2 = jnp.dot(C, h, preferred_element_type=jnp.float32)
    h = gin(h + agg2, w1b_ref[...], b1b_ref[...], g1b_ref[...],
            be1b_ref[...], w2b_ref[...], b2b_ref[...])
    ew1 = ew1_ref[...]
    g = g_ref[...]
    A = (jnp.dot(h, ew1[:DIM, :], preferred_element_type=jnp.float32)
         + eb1_ref[...])
    B = jnp.dot(h, ew1[DIM:, :], preferred_element_type=jnp.float32)
    Ac = A - jnp.mean(A, axis=-1, keepdims=True)
    Bc = B - jnp.mean(B, axis=-1, keepdims=True)
    ac_ref[...] = Ac
    bc_ref[...] = Bc
    acg_ref[...] = Ac * g
    bcg_ref[...] = Bc * g
    va_ref[...] = jnp.mean(Ac * Ac, axis=-1, keepdims=True)
    ones_row = jnp.ones((1, DIM), jnp.float32)
    vb_ref[...] = lax.dot_general(
        ones_row, Bc * Bc, (((1,), (1,)), ((), ())),
        preferred_element_type=jnp.float32) * (1.0 / DIM)


def _pair_kernel(ac_ref, bc_ref, acg_ref, bcgt_ref, va_ref, vb_ref,
                 ben_ref, w2_ref, b2_ref, o_ref):
    Ac = ac_ref[...]
    Bc = bc_ref[...]
    P = lax.dot_general(Ac, Bc, (((1,), (1,)), ((), ())),
                        preferred_element_type=jnp.float32)
    var = va_ref[...] + vb_ref[...] + (2.0 / DIM) * P
    rstd = jax.lax.rsqrt(var + 1e-5)
    t = ((acg_ref[...][:, :, None] + bcgt_ref[...][None, :, :])
         * rstd[:, None, :] + ben_ref[...][None, :, :])
    y = _gelu(t)
    w2 = w2_ref[...]
    rows = [lax.dot_general(w2, y[k], (((1,), (0,)), ((), ())),
                            preferred_element_type=jnp.float32)
            for k in range(BI)]
    o = jnp.concatenate(rows, axis=0)
    o_ref[...] = jax.nn.sigmoid(o + b2_ref[...])


def kernel(x, edge_index, c1_W1, c1_b1, c1_g1, c1_be1, c1_W2, c1_b2,
           c2_W1, c2_b1, c2_g1, c2_be1, c2_W2, c2_b2,
           e_W1, e_b1, e_g1, e_be1, e_W2, e_b2):
    ei = edge_index.astype(jnp.int32)
    src = ei[0]
    dst = ei[1]

    C = _count_sc(src, dst).reshape(_SC_CORES, N, N)

    r = lambda v: v.reshape(1, -1)
    Ac, Bc, Acg, Bcg, Va, Vb = pl.pallas_call(
        _mlp_kernel,
        out_shape=(jax.ShapeDtypeStruct((N, DIM), jnp.float32),
                   jax.ShapeDtypeStruct((N, DIM), jnp.float32),
                   jax.ShapeDtypeStruct((N, DIM), jnp.float32),
                   jax.ShapeDtypeStruct((N, DIM), jnp.float32),
                   jax.ShapeDtypeStruct((N, 1), jnp.float32),
                   jax.ShapeDtypeStruct((1, N), jnp.float32)),
    )(C, x,
      c1_W1, r(c1_b1), r(c1_g1), r(c1_be1), c1_W2, r(c1_b2),
      c2_W1, r(c2_b1), r(c2_g1), r(c2_be1), c2_W2, r(c2_b2),
      e_W1, r(e_b1), r(e_g1))

    BcgT = Bcg.T
    beN = jnp.broadcast_to(e_be1.reshape(DIM, 1), (DIM, N))
    out = pl.pallas_call(
        _pair_kernel,
        grid=(N // BI,),
        in_specs=[pl.BlockSpec((BI, DIM), lambda i: (i, 0)),
                  pl.BlockSpec((N, DIM), lambda i: (0, 0)),
                  pl.BlockSpec((BI, DIM), lambda i: (i, 0)),
                  pl.BlockSpec((DIM, N), lambda i: (0, 0)),
                  pl.BlockSpec((BI, 1), lambda i: (i, 0)),
                  pl.BlockSpec((1, N), lambda i: (0, 0)),
                  pl.BlockSpec((DIM, N), lambda i: (0, 0)),
                  pl.BlockSpec((1, DIM), lambda i: (0, 0)),
                  pl.BlockSpec((1, 1), lambda i: (0, 0))],
        out_specs=pl.BlockSpec((BI, N), lambda i: (i, 0)),
        out_shape=jax.ShapeDtypeStruct((N, N), jnp.float32),
    )(Ac, Bc, Acg, BcgT, Va, Vb, beN, e_W2.reshape(1, DIM),
      e_b2.reshape(1, 1))
    return out


# speedup vs baseline: 2.2657x; 1.0516x over previous
import functools

import jax
import jax.numpy as jnp
from jax import lax
from jax.experimental import pallas as pl
from jax.experimental.pallas import tpu as pltpu
from jax.experimental.pallas import tpu_sc as plsc

N = 512
DIN = 512
DIM = 128
E = 16384
BI = 16

_SC_CORES = 2
_SC_SUBCORES = 16
_ROWS_PER_W = N // _SC_SUBCORES
_HE = E // _SC_CORES
_LANES = 16


def _count_sc_body(src_hbm, dst_hbm, out_hbm, src_v, dst_v, c_v):
    cid = lax.axis_index("c")
    sid = lax.axis_index("s")
    lo = sid * _ROWS_PER_W
    pltpu.sync_copy(src_hbm.at[pl.ds(cid * _HE, _HE)], src_v)
    pltpu.sync_copy(dst_hbm.at[pl.ds(cid * _HE, _HE)], dst_v)
    zeros = jnp.zeros((_LANES,), jnp.float32)

    @plsc.parallel_loop(0, _ROWS_PER_W * N // _LANES, unroll=8)
    def _zero(j):
        c_v[pl.ds(j * _LANES, _LANES)] = zeros

    @plsc.parallel_loop(0, _HE // _LANES, unroll=4)
    def _accum(i):
        d = dst_v[pl.ds(i * _LANES, _LANES)]
        s = src_v[pl.ds(i * _LANES, _LANES)]
        m = (d >= lo) & (d < lo + _ROWS_PER_W)
        idx = (d - lo) * N + s
        cnt, last = plsc.scan_count(idx, m)
        plsc.addupdate_scatter(c_v, [idx], cnt.astype(jnp.float32),
                               mask=last)

    pltpu.sync_copy(
        c_v, out_hbm.at[pl.ds((cid * N + lo) * N, _ROWS_PER_W * N)])


_count_sc = functools.partial(
    pl.kernel,
    out_type=jax.ShapeDtypeStruct((_SC_CORES * N * N,), jnp.float32),
    mesh=plsc.VectorSubcoreMesh(core_axis_name="c", subcore_axis_name="s"),
    compiler_params=pltpu.CompilerParams(needs_layout_passes=False),
    scratch_types=[
        pltpu.VMEM((_HE,), jnp.int32),
        pltpu.VMEM((_HE,), jnp.int32),
        pltpu.VMEM((_ROWS_PER_W * N,), jnp.float32),
    ],
)(_count_sc_body)


def _ln(t, g, b, eps=1e-5):
    mu = jnp.mean(t, axis=-1, keepdims=True)
    d = t - mu
    var = jnp.mean(d * d, axis=-1, keepdims=True)
    return d * jax.lax.rsqrt(var + eps) * g + b


def _gelu(t):
    return 0.5 * t * (1.0 + jax.lax.erf(t * 0.7071067811865476))


def _mlp_kernel(c_ref, x_ref,
                w1a_ref, b1a_ref, g1a_ref, be1a_ref, w2a_ref, b2a_ref,
                w1b_ref, b1b_ref, g1b_ref, be1b_ref, w2b_ref, b2b_ref,
                ew1_ref, eb1_ref, g_ref,
                ac_ref, bc_ref, acg_ref, bcg_ref, va_ref, vb_ref):
    C = c_ref[0] + c_ref[1]
    x = x_ref[...]

    def gin(h, w1, b1, g1, be1, w2, b2):
        t = jnp.dot(h, w1, preferred_element_type=jnp.float32) + b1
        t = _gelu(_ln(t, g1, be1))
        return jnp.dot(t, w2, preferred_element_type=jnp.float32) + b2

    agg1 = jnp.dot(C, x, preferred_element_type=jnp.float32)
    h = gin(x + agg1, w1a_ref[...], b1a_ref[...], g1a_ref[...],
            be1a_ref[...], w2a_ref[...], b2a_ref[...])
    agg2 = jnp.dot(C, h, preferred_element_type=jnp.float32)
    h = gin(h + agg2, w1b_ref[...], b1b_ref[...], g1b_ref[...],
            be1b_ref[...], w2b_ref[...], b2b_ref[...])
    ew1 = ew1_ref[...]
    g = g_ref[...]
    A = (jnp.dot(h, ew1[:DIM, :], preferred_element_type=jnp.float32)
         + eb1_ref[...])
    B = jnp.dot(h, ew1[DIM:, :], preferred_element_type=jnp.float32)
    Ac = A - jnp.mean(A, axis=-1, keepdims=True)
    Bc = B - jnp.mean(B, axis=-1, keepdims=True)
    ac_ref[...] = Ac
    bc_ref[...] = Bc
    acg_ref[...] = Ac * g
    bcg_ref[...] = Bc * g
    va_ref[...] = jnp.mean(Ac * Ac, axis=-1, keepdims=True)
    ones_row = jnp.ones((1, DIM), jnp.float32)
    vb_ref[...] = lax.dot_general(
        ones_row, Bc * Bc, (((1,), (1,)), ((), ())),
        preferred_element_type=jnp.float32) * (1.0 / DIM)


def _pair_kernel(ac_ref, bc_ref, acg_ref, bcgt_ref, va_ref, vb_ref,
                 ben_ref, w2_ref, b2_ref, o_ref):
    Ac = ac_ref[...]
    Bc = bc_ref[...]
    P = lax.dot_general(Ac, Bc, (((1,), (1,)), ((), ())),
                        preferred_element_type=jnp.float32)
    var = va_ref[...] + vb_ref[...] + (2.0 / DIM) * P
    rstd = jax.lax.rsqrt(var + 1e-5)
    c = 0.7071067811865476
    rstd_s = rstd * c
    s = ((acg_ref[...][:, :, None] + bcgt_ref[...][None, :, :])
         * rstd_s[:, None, :] + ben_ref[...][None, :, :])
    u = s * c
    y = u + u * jax.lax.erf(s)
    w2 = w2_ref[...]
    rows = [lax.dot_general(w2, y[k], (((1,), (0,)), ((), ())),
                            preferred_element_type=jnp.float32)
            for k in range(BI)]
    o = jnp.concatenate(rows, axis=0)
    o_ref[...] = jax.nn.sigmoid(o + b2_ref[...])


def kernel(x, edge_index, c1_W1, c1_b1, c1_g1, c1_be1, c1_W2, c1_b2,
           c2_W1, c2_b1, c2_g1, c2_be1, c2_W2, c2_b2,
           e_W1, e_b1, e_g1, e_be1, e_W2, e_b2):
    ei = edge_index.astype(jnp.int32)
    src = ei[0]
    dst = ei[1]

    C = _count_sc(src, dst).reshape(_SC_CORES, N, N)

    r = lambda v: v.reshape(1, -1)
    Ac, Bc, Acg, Bcg, Va, Vb = pl.pallas_call(
        _mlp_kernel,
        out_shape=(jax.ShapeDtypeStruct((N, DIM), jnp.float32),
                   jax.ShapeDtypeStruct((N, DIM), jnp.float32),
                   jax.ShapeDtypeStruct((N, DIM), jnp.float32),
                   jax.ShapeDtypeStruct((N, DIM), jnp.float32),
                   jax.ShapeDtypeStruct((N, 1), jnp.float32),
                   jax.ShapeDtypeStruct((1, N), jnp.float32)),
    )(C, x,
      c1_W1, r(c1_b1), r(c1_g1), r(c1_be1), c1_W2, r(c1_b2),
      c2_W1, r(c2_b1), r(c2_g1), r(c2_be1), c2_W2, r(c2_b2),
      e_W1, r(e_b1), r(e_g1))

    BcgT = Bcg.T
    beN = jnp.broadcast_to(
        (e_be1 * 0.7071067811865476).reshape(DIM, 1), (DIM, N))
    out = pl.pallas_call(
        _pair_kernel,
        grid=(N // BI,),
        in_specs=[pl.BlockSpec((BI, DIM), lambda i: (i, 0)),
                  pl.BlockSpec((N, DIM), lambda i: (0, 0)),
                  pl.BlockSpec((BI, DIM), lambda i: (i, 0)),
                  pl.BlockSpec((DIM, N), lambda i: (0, 0)),
                  pl.BlockSpec((BI, 1), lambda i: (i, 0)),
                  pl.BlockSpec((1, N), lambda i: (0, 0)),
                  pl.BlockSpec((DIM, N), lambda i: (0, 0)),
                  pl.BlockSpec((1, DIM), lambda i: (0, 0)),
                  pl.BlockSpec((1, 1), lambda i: (0, 0))],
        out_specs=pl.BlockSpec((BI, N), lambda i: (i, 0)),
        out_shape=jax.ShapeDtypeStruct((N, N), jnp.float32),
    )(Ac, Bc, Acg, BcgT, Va, Vb, beN, e_W2.reshape(1, DIM),
      e_b2.reshape(1, 1))
    return out


# speedup vs baseline: 2.3948x; 1.0570x over previous
import functools

import jax
import jax.numpy as jnp
from jax import lax
from jax.experimental import pallas as pl
from jax.experimental.pallas import tpu as pltpu
from jax.experimental.pallas import tpu_sc as plsc

N = 512
DIN = 512
DIM = 128
E = 16384
BI = 16

_SC_CORES = 2
_SC_SUBCORES = 16
_ROWS_PER_W = N // _SC_SUBCORES
_HE = E // _SC_CORES
_LANES = 16


def _count_sc_body(src_hbm, dst_hbm, out_hbm, src_v, dst_v, c_v):
    cid = lax.axis_index("c")
    sid = lax.axis_index("s")
    lo = sid * _ROWS_PER_W
    pltpu.sync_copy(src_hbm.at[pl.ds(cid * _HE, _HE)], src_v)
    pltpu.sync_copy(dst_hbm.at[pl.ds(cid * _HE, _HE)], dst_v)
    zeros = jnp.zeros((_LANES,), jnp.float32)

    @plsc.parallel_loop(0, _ROWS_PER_W * N // _LANES, unroll=8)
    def _zero(j):
        c_v[pl.ds(j * _LANES, _LANES)] = zeros

    @plsc.parallel_loop(0, _HE // _LANES, unroll=4)
    def _accum(i):
        d = dst_v[pl.ds(i * _LANES, _LANES)]
        s = src_v[pl.ds(i * _LANES, _LANES)]
        m = (d >= lo) & (d < lo + _ROWS_PER_W)
        idx = (d - lo) * N + s
        cnt, last = plsc.scan_count(idx, m)
        plsc.addupdate_scatter(c_v, [idx], cnt.astype(jnp.float32),
                               mask=last)

    pltpu.sync_copy(
        c_v, out_hbm.at[pl.ds((cid * N + lo) * N, _ROWS_PER_W * N)])


_count_sc = functools.partial(
    pl.kernel,
    out_type=jax.ShapeDtypeStruct((_SC_CORES * N * N,), jnp.float32),
    mesh=plsc.VectorSubcoreMesh(core_axis_name="c", subcore_axis_name="s"),
    compiler_params=pltpu.CompilerParams(needs_layout_passes=False),
    scratch_types=[
        pltpu.VMEM((_HE,), jnp.int32),
        pltpu.VMEM((_HE,), jnp.int32),
        pltpu.VMEM((_ROWS_PER_W * N,), jnp.float32),
    ],
)(_count_sc_body)


def _ln(t, g, b, eps=1e-5):
    mu = jnp.mean(t, axis=-1, keepdims=True)
    d = t - mu
    var = jnp.mean(d * d, axis=-1, keepdims=True)
    return d * jax.lax.rsqrt(var + eps) * g + b


def _gelu(t):
    return 0.5 * t * (1.0 + jax.lax.erf(t * 0.7071067811865476))


def _fused_kernel(c_ref, x_ref,
                  w1a_ref, b1a_ref, g1a_ref, be1a_ref, w2a_ref, b2a_ref,
                  w1b_ref, b1b_ref, g1b_ref, be1b_ref, w2b_ref, b2b_ref,
                  ew1_ref, eb1_ref, g_ref, bes_ref, w2_ref, b2_ref,
                  o_ref,
                  ac_s, bct_s, acg_s, bcgt_s, va_s, vb_s, ben_s):
    i = pl.program_id(0)
    c = 0.7071067811865476

    @pl.when(i == 0)
    def _():
        C = c_ref[0] + c_ref[1]
        x = x_ref[...]

        def gin(h, w1, b1, g1, be1, w2, b2):
            t = jnp.dot(h, w1, preferred_element_type=jnp.float32) + b1
            t = _gelu(_ln(t, g1, be1))
            return jnp.dot(t, w2, preferred_element_type=jnp.float32) + b2

        agg1 = jnp.dot(C, x, preferred_element_type=jnp.float32)
        h = gin(x + agg1, w1a_ref[...], b1a_ref[...], g1a_ref[...],
                be1a_ref[...], w2a_ref[...], b2a_ref[...])
        agg2 = jnp.dot(C, h, preferred_element_type=jnp.float32)
        h = gin(h + agg2, w1b_ref[...], b1b_ref[...], g1b_ref[...],
                be1b_ref[...], w2b_ref[...], b2b_ref[...])
        ew1 = ew1_ref[...]
        g = g_ref[...]
        A = (jnp.dot(h, ew1[:DIM, :], preferred_element_type=jnp.float32)
             + eb1_ref[...])
        B = jnp.dot(h, ew1[DIM:, :], preferred_element_type=jnp.float32)
        Ac = A - jnp.mean(A, axis=-1, keepdims=True)
        va_s[...] = jnp.mean(Ac * Ac, axis=-1, keepdims=True)
        ac_s[...] = Ac
        acg_s[...] = Ac * g
        eye = jnp.eye(DIM, dtype=jnp.float32)
        BT = lax.dot_general(eye, B, (((1,), (1,)), ((), ())),
                             preferred_element_type=jnp.float32)
        ones_row = jnp.ones((1, DIM), jnp.float32)
        mB = lax.dot_general(ones_row, B, (((1,), (1,)), ((), ())),
                             preferred_element_type=jnp.float32) / DIM
        BcT = BT - mB
        bct_s[...] = BcT
        vb_s[...] = jnp.sum(BcT * BcT, axis=0, keepdims=True) / DIM
        bcgt_s[...] = BcT * g_ref[...].reshape(DIM, 1)
        ben_s[...] = jnp.broadcast_to(bes_ref[...], (DIM, N))

    ac = ac_s[pl.ds(i * BI, BI), :]
    acg = acg_s[pl.ds(i * BI, BI), :]
    va = va_s[pl.ds(i * BI, BI), :]
    bcgt = bcgt_s[...]
    P = lax.dot_general(ac, bct_s[...], (((1,), (0,)), ((), ())),
                        preferred_element_type=jnp.float32)
    var = va + vb_s[...] + (2.0 / DIM) * P
    rstd = jax.lax.rsqrt(var + 1e-5) * c
    s = ((acg[:, :, None] + bcgt[None, :, :]) * rstd[:, None, :]
         + ben_s[...][None, :, :])
    u = s * c
    y = u + u * jax.lax.erf(s)
    w2 = w2_ref[...]
    rows = [lax.dot_general(w2, y[k], (((1,), (0,)), ((), ())),
                            preferred_element_type=jnp.float32)
            for k in range(BI)]
    o = jnp.concatenate(rows, axis=0)
    o_ref[...] = jax.nn.sigmoid(o + b2_ref[...])


def kernel(x, edge_index, c1_W1, c1_b1, c1_g1, c1_be1, c1_W2, c1_b2,
           c2_W1, c2_b1, c2_g1, c2_be1, c2_W2, c2_b2,
           e_W1, e_b1, e_g1, e_be1, e_W2, e_b2):
    ei = edge_index.astype(jnp.int32)
    src = ei[0]
    dst = ei[1]

    C = _count_sc(src, dst).reshape(_SC_CORES, N, N)

    r = lambda v: v.reshape(1, -1)
    full = lambda shape: pl.BlockSpec(shape, lambda i: tuple(0 for _ in shape))
    bes = (e_be1 * 0.7071067811865476).reshape(DIM, 1)
    out = pl.pallas_call(
        _fused_kernel,
        grid=(N // BI,),
        in_specs=[full((_SC_CORES, N, N)), full((N, DIN)),
                  full((DIN, DIM)), full((1, DIM)), full((1, DIM)),
                  full((1, DIM)), full((DIM, DIM)), full((1, DIM)),
                  full((DIM, DIM)), full((1, DIM)), full((1, DIM)),
                  full((1, DIM)), full((DIM, DIM)), full((1, DIM)),
                  full((2 * DIM, DIM)), full((1, DIM)), full((1, DIM)),
                  full((DIM, 1)), full((1, DIM)), full((1, 1))],
        out_specs=pl.BlockSpec((BI, N), lambda i: (i, 0)),
        out_shape=jax.ShapeDtypeStruct((N, N), jnp.float32),
        scratch_shapes=[pltpu.VMEM((N, DIM), jnp.float32),
                        pltpu.VMEM((DIM, N), jnp.float32),
                        pltpu.VMEM((N, DIM), jnp.float32),
                        pltpu.VMEM((DIM, N), jnp.float32),
                        pltpu.VMEM((N, 1), jnp.float32),
                        pltpu.VMEM((1, N), jnp.float32),
                        pltpu.VMEM((DIM, N), jnp.float32)],
    )(C, x,
      c1_W1, r(c1_b1), r(c1_g1), r(c1_be1), c1_W2, r(c1_b2),
      c2_W1, r(c2_b1), r(c2_g1), r(c2_be1), c2_W2, r(c2_b2),
      e_W1, r(e_b1), r(e_g1), bes, e_W2.reshape(1, DIM),
      e_b2.reshape(1, 1))
    return out


# speedup vs baseline: 2.5177x; 1.0513x over previous
import functools

import jax
import jax.numpy as jnp
from jax import lax
from jax.experimental import pallas as pl
from jax.experimental.pallas import tpu as pltpu
from jax.experimental.pallas import tpu_sc as plsc

N = 512
DIN = 512
DIM = 128
E = 16384
BI = 16

_SC_CORES = 2
_SC_SUBCORES = 16
_ROWS_PER_W = N // _SC_SUBCORES
_HE = E // _SC_CORES
_LANES = 16


def _count_sc_body(src_hbm, dst_hbm, out_hbm, src_v, dst_v, c_v):
    cid = lax.axis_index("c")
    sid = lax.axis_index("s")
    lo = sid * _ROWS_PER_W
    pltpu.sync_copy(src_hbm.at[pl.ds(cid * _HE, _HE)], src_v)
    pltpu.sync_copy(dst_hbm.at[pl.ds(cid * _HE, _HE)], dst_v)
    zeros = jnp.zeros((_LANES,), jnp.float32)

    @plsc.parallel_loop(0, _ROWS_PER_W * N // _LANES, unroll=8)
    def _zero(j):
        c_v[pl.ds(j * _LANES, _LANES)] = zeros

    @plsc.parallel_loop(0, _HE // _LANES, unroll=4)
    def _accum(i):
        d = dst_v[pl.ds(i * _LANES, _LANES)]
        s = src_v[pl.ds(i * _LANES, _LANES)]
        m = (d >= lo) & (d < lo + _ROWS_PER_W)
        idx = (d - lo) * N + s
        cnt, last = plsc.scan_count(idx, m)
        plsc.addupdate_scatter(c_v, [idx], cnt.astype(jnp.float32),
                               mask=last)

    pltpu.sync_copy(
        c_v, out_hbm.at[pl.ds((cid * N + lo) * N, _ROWS_PER_W * N)])


_count_sc = functools.partial(
    pl.kernel,
    out_type=jax.ShapeDtypeStruct((_SC_CORES * N * N,), jnp.float32),
    mesh=plsc.VectorSubcoreMesh(core_axis_name="c", subcore_axis_name="s"),
    compiler_params=pltpu.CompilerParams(needs_layout_passes=False),
    scratch_types=[
        pltpu.VMEM((_HE,), jnp.int32),
        pltpu.VMEM((_HE,), jnp.int32),
        pltpu.VMEM((_ROWS_PER_W * N,), jnp.float32),
    ],
)(_count_sc_body)


def _ln(t, g, b, eps=1e-5):
    mu = jnp.mean(t, axis=-1, keepdims=True)
    d = t - mu
    var = jnp.mean(d * d, axis=-1, keepdims=True)
    return d * jax.lax.rsqrt(var + eps) * g + b


def _gelu(t):
    return 0.5 * t * (1.0 + jax.lax.erf(t * 0.7071067811865476))


def _fused_kernel(c_ref, x_ref,
                  w1a_ref, b1a_ref, g1a_ref, be1a_ref, w2a_ref, b2a_ref,
                  w1b_ref, b1b_ref, g1b_ref, be1b_ref, w2b_ref, b2b_ref,
                  ew1_ref, eb1_ref, g_ref, bes_ref, w2_ref, b2_ref,
                  o_ref,
                  ac_s, bct_s, acg_s, bcgt_s, va_s, vb_s, ben_s):
    i = pl.program_id(0)
    c = 0.7071067811865476

    @pl.when(i == 0)
    def _():
        C = c_ref[0] + c_ref[1]
        x = x_ref[...]

        def gin(h, w1, b1, g1, be1, w2, b2):
            t = jnp.dot(h, w1, preferred_element_type=jnp.float32) + b1
            t = _gelu(_ln(t, g1, be1))
            return jnp.dot(t, w2, preferred_element_type=jnp.float32) + b2

        agg1 = jnp.dot(C, x, preferred_element_type=jnp.float32)
        h = gin(x + agg1, w1a_ref[...], b1a_ref[...], g1a_ref[...],
                be1a_ref[...], w2a_ref[...], b2a_ref[...])
        agg2 = jnp.dot(C, h, preferred_element_type=jnp.float32)
        h = gin(h + agg2, w1b_ref[...], b1b_ref[...], g1b_ref[...],
                be1b_ref[...], w2b_ref[...], b2b_ref[...])
        ew1 = ew1_ref[...]
        g = g_ref[...]
        A = (jnp.dot(h, ew1[:DIM, :], preferred_element_type=jnp.float32)
             + eb1_ref[...])
        B = jnp.dot(h, ew1[DIM:, :], preferred_element_type=jnp.float32)
        Ac = A - jnp.mean(A, axis=-1, keepdims=True)
        va_s[...] = jnp.mean(Ac * Ac, axis=-1, keepdims=True)
        ac_s[...] = Ac
        acg_s[...] = Ac * g
        eye = jnp.eye(DIM, dtype=jnp.float32)
        BT = lax.dot_general(eye, B, (((1,), (1,)), ((), ())),
                             preferred_element_type=jnp.float32)
        ones_row = jnp.ones((1, DIM), jnp.float32)
        mB = lax.dot_general(ones_row, B, (((1,), (1,)), ((), ())),
                             preferred_element_type=jnp.float32) / DIM
        BcT = BT - mB
        bct_s[...] = BcT
        vb_s[...] = jnp.sum(BcT * BcT, axis=0, keepdims=True) / DIM
        bcgt_s[...] = BcT * g_ref[...].reshape(DIM, 1)
        ben_s[...] = jnp.broadcast_to(bes_ref[...], (DIM, N))

    ac = ac_s[pl.ds(i * BI, BI), :]
    acg = acg_s[pl.ds(i * BI, BI), :]
    va = va_s[pl.ds(i * BI, BI), :]
    bcgt = bcgt_s[...]
    P = lax.dot_general(ac, bct_s[...], (((1,), (0,)), ((), ())),
                        preferred_element_type=jnp.float32)
    var = va + vb_s[...] + (2.0 / DIM) * P
    rstd = jax.lax.rsqrt(var + 1e-5) * c
    s = ((acg[:, :, None] + bcgt[None, :, :]) * rstd[:, None, :]
         + ben_s[...][None, :, :])
    y = s + s * jax.lax.erf(s)
    w2 = w2_ref[...]
    rows = [lax.dot_general(w2, y[k], (((1,), (0,)), ((), ())),
                            preferred_element_type=jnp.float32)
            for k in range(BI)]
    o = jnp.concatenate(rows, axis=0)
    o_ref[...] = jax.nn.sigmoid(o + b2_ref[...])


def kernel(x, edge_index, c1_W1, c1_b1, c1_g1, c1_be1, c1_W2, c1_b2,
           c2_W1, c2_b1, c2_g1, c2_be1, c2_W2, c2_b2,
           e_W1, e_b1, e_g1, e_be1, e_W2, e_b2):
    ei = edge_index.astype(jnp.int32)
    src = ei[0]
    dst = ei[1]

    C = _count_sc(src, dst).reshape(_SC_CORES, N, N)

    r = lambda v: v.reshape(1, -1)
    full = lambda shape: pl.BlockSpec(shape, lambda i: tuple(0 for _ in shape))
    bes = (e_be1 * 0.7071067811865476).reshape(DIM, 1)
    out = pl.pallas_call(
        _fused_kernel,
        grid=(N // BI,),
        in_specs=[full((_SC_CORES, N, N)), full((N, DIN)),
                  full((DIN, DIM)), full((1, DIM)), full((1, DIM)),
                  full((1, DIM)), full((DIM, DIM)), full((1, DIM)),
                  full((DIM, DIM)), full((1, DIM)), full((1, DIM)),
                  full((1, DIM)), full((DIM, DIM)), full((1, DIM)),
                  full((2 * DIM, DIM)), full((1, DIM)), full((1, DIM)),
                  full((DIM, 1)), full((1, DIM)), full((1, 1))],
        out_specs=pl.BlockSpec((BI, N), lambda i: (i, 0)),
        out_shape=jax.ShapeDtypeStruct((N, N), jnp.float32),
        scratch_shapes=[pltpu.VMEM((N, DIM), jnp.float32),
                        pltpu.VMEM((DIM, N), jnp.float32),
                        pltpu.VMEM((N, DIM), jnp.float32),
                        pltpu.VMEM((DIM, N), jnp.float32),
                        pltpu.VMEM((N, 1), jnp.float32),
                        pltpu.VMEM((1, N), jnp.float32),
                        pltpu.VMEM((DIM, N), jnp.float32)],
    )(C, x,
      c1_W1, r(c1_b1), r(c1_g1), r(c1_be1), c1_W2, r(c1_b2),
      c2_W1, r(c2_b1), r(c2_g1), r(c2_be1), c2_W2, r(c2_b2),
      e_W1, r(e_b1), r(e_g1), bes,
      (e_W2 * 0.7071067811865476).reshape(1, DIM),
      e_b2.reshape(1, 1))
    return out


# speedup vs baseline: 2.6861x; 1.0669x over previous
import functools

import jax
import jax.numpy as jnp
from jax import lax
from jax.experimental import pallas as pl
from jax.experimental.pallas import tpu as pltpu
from jax.experimental.pallas import tpu_sc as plsc

N = 512
DIN = 512
DIM = 128
E = 16384
BI = 32

_SC_CORES = 2
_SC_SUBCORES = 16
_ROWS_PER_W = N // _SC_SUBCORES
_HE = E // _SC_CORES
_LANES = 16


def _count_sc_body(src_hbm, dst_hbm, out_hbm, src_v, dst_v, c_v):
    cid = lax.axis_index("c")
    sid = lax.axis_index("s")
    lo = sid * _ROWS_PER_W
    pltpu.sync_copy(src_hbm.at[pl.ds(cid * _HE, _HE)], src_v)
    pltpu.sync_copy(dst_hbm.at[pl.ds(cid * _HE, _HE)], dst_v)
    zeros = jnp.zeros((_LANES,), jnp.float32)

    @plsc.parallel_loop(0, _ROWS_PER_W * N // _LANES, unroll=8)
    def _zero(j):
        c_v[pl.ds(j * _LANES, _LANES)] = zeros

    @plsc.parallel_loop(0, _HE // _LANES, unroll=4)
    def _accum(i):
        d = dst_v[pl.ds(i * _LANES, _LANES)]
        s = src_v[pl.ds(i * _LANES, _LANES)]
        m = (d >= lo) & (d < lo + _ROWS_PER_W)
        idx = (d - lo) * N + s
        cnt, last = plsc.scan_count(idx, m)
        plsc.addupdate_scatter(c_v, [idx], cnt.astype(jnp.float32),
                               mask=last)

    pltpu.sync_copy(
        c_v, out_hbm.at[pl.ds((cid * N + lo) * N, _ROWS_PER_W * N)])


_count_sc = functools.partial(
    pl.kernel,
    out_type=jax.ShapeDtypeStruct((_SC_CORES * N * N,), jnp.float32),
    mesh=plsc.VectorSubcoreMesh(core_axis_name="c", subcore_axis_name="s"),
    compiler_params=pltpu.CompilerParams(needs_layout_passes=False),
    scratch_types=[
        pltpu.VMEM((_HE,), jnp.int32),
        pltpu.VMEM((_HE,), jnp.int32),
        pltpu.VMEM((_ROWS_PER_W * N,), jnp.float32),
    ],
)(_count_sc_body)


def _ln(t, g, b, eps=1e-5):
    mu = jnp.mean(t, axis=-1, keepdims=True)
    d = t - mu
    var = jnp.mean(d * d, axis=-1, keepdims=True)
    return d * jax.lax.rsqrt(var + eps) * g + b


def _gelu(t):
    return 0.5 * t * (1.0 + jax.lax.erf(t * 0.7071067811865476))


def _fused_kernel(c_ref, x_ref,
                  w1a_ref, b1a_ref, g1a_ref, be1a_ref, w2a_ref, b2a_ref,
                  w1b_ref, b1b_ref, g1b_ref, be1b_ref, w2b_ref, b2b_ref,
                  ew1_ref, eb1_ref, g_ref, bes_ref, w2_ref, b2_ref,
                  o_ref,
                  ac_s, bct_s, acg_s, bcgt_s, va_s, vb_s, ben_s):
    i = pl.program_id(0)
    c = 0.7071067811865476

    @pl.when(i == 0)
    def _():
        C = c_ref[0] + c_ref[1]
        x = x_ref[...]

        def gin(h, w1, b1, g1, be1, w2, b2):
            t = jnp.dot(h, w1, preferred_element_type=jnp.float32) + b1
            t = _gelu(_ln(t, g1, be1))
            return jnp.dot(t, w2, preferred_element_type=jnp.float32) + b2

        agg1 = jnp.dot(C, x, preferred_element_type=jnp.float32)
        h = gin(x + agg1, w1a_ref[...], b1a_ref[...], g1a_ref[...],
                be1a_ref[...], w2a_ref[...], b2a_ref[...])
        agg2 = jnp.dot(C, h, preferred_element_type=jnp.float32)
        h = gin(h + agg2, w1b_ref[...], b1b_ref[...], g1b_ref[...],
                be1b_ref[...], w2b_ref[...], b2b_ref[...])
        ew1 = ew1_ref[...]
        g = g_ref[...]
        A = (jnp.dot(h, ew1[:DIM, :], preferred_element_type=jnp.float32)
             + eb1_ref[...])
        B = jnp.dot(h, ew1[DIM:, :], preferred_element_type=jnp.float32)
        Ac = A - jnp.mean(A, axis=-1, keepdims=True)
        va_s[...] = jnp.mean(Ac * Ac, axis=-1, keepdims=True)
        ac_s[...] = Ac
        acg_s[...] = Ac * g
        eye = jnp.eye(DIM, dtype=jnp.float32)
        BT = lax.dot_general(eye, B, (((1,), (1,)), ((), ())),
                             preferred_element_type=jnp.float32)
        ones_row = jnp.ones((1, DIM), jnp.float32)
        mB = lax.dot_general(ones_row, B, (((1,), (1,)), ((), ())),
                             preferred_element_type=jnp.float32) / DIM
        BcT = BT - mB
        bct_s[...] = BcT
        vb_s[...] = jnp.sum(BcT * BcT, axis=0, keepdims=True) / DIM
        bcgt_s[...] = BcT * g_ref[...].reshape(DIM, 1)
        ben_s[...] = jnp.broadcast_to(bes_ref[...], (DIM, N))

    ac = ac_s[pl.ds(i * BI, BI), :]
    acg = acg_s[pl.ds(i * BI, BI), :]
    va = va_s[pl.ds(i * BI, BI), :]
    bcgt = bcgt_s[...]
    P = lax.dot_general(ac, bct_s[...], (((1,), (0,)), ((), ())),
                        preferred_element_type=jnp.float32)
    var = va + vb_s[...] + (2.0 / DIM) * P
    rstd = jax.lax.rsqrt(var + 1e-5) * c
    s = ((acg[:, :, None] + bcgt[None, :, :]) * rstd[:, None, :]
         + ben_s[...][None, :, :])
    y = s + s * jax.lax.erf(s)
    w2 = w2_ref[...]
    rows = [lax.dot_general(w2, y[k], (((1,), (0,)), ((), ())),
                            preferred_element_type=jnp.float32)
            for k in range(BI)]
    o = jnp.concatenate(rows, axis=0)
    o_ref[...] = jax.nn.sigmoid(o + b2_ref[...])


def kernel(x, edge_index, c1_W1, c1_b1, c1_g1, c1_be1, c1_W2, c1_b2,
           c2_W1, c2_b1, c2_g1, c2_be1, c2_W2, c2_b2,
           e_W1, e_b1, e_g1, e_be1, e_W2, e_b2):
    ei = edge_index.astype(jnp.int32)
    src = ei[0]
    dst = ei[1]

    C = _count_sc(src, dst).reshape(_SC_CORES, N, N)

    r = lambda v: v.reshape(1, -1)
    full = lambda shape: pl.BlockSpec(shape, lambda i: tuple(0 for _ in shape))
    bes = (e_be1 * 0.7071067811865476).reshape(DIM, 1)
    out = pl.pallas_call(
        _fused_kernel,
        grid=(N // BI,),
        in_specs=[full((_SC_CORES, N, N)), full((N, DIN)),
                  full((DIN, DIM)), full((1, DIM)), full((1, DIM)),
                  full((1, DIM)), full((DIM, DIM)), full((1, DIM)),
                  full((DIM, DIM)), full((1, DIM)), full((1, DIM)),
                  full((1, DIM)), full((DIM, DIM)), full((1, DIM)),
                  full((2 * DIM, DIM)), full((1, DIM)), full((1, DIM)),
                  full((DIM, 1)), full((1, DIM)), full((1, 1))],
        out_specs=pl.BlockSpec((BI, N), lambda i: (i, 0)),
        out_shape=jax.ShapeDtypeStruct((N, N), jnp.float32),
        scratch_shapes=[pltpu.VMEM((N, DIM), jnp.float32),
                        pltpu.VMEM((DIM, N), jnp.float32),
                        pltpu.VMEM((N, DIM), jnp.float32),
                        pltpu.VMEM((DIM, N), jnp.float32),
                        pltpu.VMEM((N, 1), jnp.float32),
                        pltpu.VMEM((1, N), jnp.float32),
                        pltpu.VMEM((DIM, N), jnp.float32)],
    )(C, x,
      c1_W1, r(c1_b1), r(c1_g1), r(c1_be1), c1_W2, r(c1_b2),
      c2_W1, r(c2_b1), r(c2_g1), r(c2_be1), c2_W2, r(c2_b2),
      e_W1, r(e_b1), r(e_g1), bes,
      (e_W2 * 0.7071067811865476).reshape(1, DIM),
      e_b2.reshape(1, 1))
    return out


# speedup vs baseline: 2.7854x; 1.0370x over previous
import functools

import jax
import jax.numpy as jnp
from jax import lax
from jax.experimental import pallas as pl
from jax.experimental.pallas import tpu as pltpu
from jax.experimental.pallas import tpu_sc as plsc

N = 512
DIN = 512
DIM = 128
E = 16384
BI = 64

_SC_CORES = 2
_SC_SUBCORES = 16
_ROWS_PER_W = N // _SC_SUBCORES
_HE = E // _SC_CORES
_LANES = 16


def _count_sc_body(src_hbm, dst_hbm, out_hbm, src_v, dst_v, c_v):
    cid = lax.axis_index("c")
    sid = lax.axis_index("s")
    lo = sid * _ROWS_PER_W
    pltpu.sync_copy(src_hbm.at[pl.ds(cid * _HE, _HE)], src_v)
    pltpu.sync_copy(dst_hbm.at[pl.ds(cid * _HE, _HE)], dst_v)
    zeros = jnp.zeros((_LANES,), jnp.float32)

    @plsc.parallel_loop(0, _ROWS_PER_W * N // _LANES, unroll=8)
    def _zero(j):
        c_v[pl.ds(j * _LANES, _LANES)] = zeros

    @plsc.parallel_loop(0, _HE // _LANES, unroll=4)
    def _accum(i):
        d = dst_v[pl.ds(i * _LANES, _LANES)]
        s = src_v[pl.ds(i * _LANES, _LANES)]
        m = (d >= lo) & (d < lo + _ROWS_PER_W)
        idx = (d - lo) * N + s
        cnt, last = plsc.scan_count(idx, m)
        plsc.addupdate_scatter(c_v, [idx], cnt.astype(jnp.float32),
                               mask=last)

    pltpu.sync_copy(
        c_v, out_hbm.at[pl.ds((cid * N + lo) * N, _ROWS_PER_W * N)])


_count_sc = functools.partial(
    pl.kernel,
    out_type=jax.ShapeDtypeStruct((_SC_CORES * N * N,), jnp.float32),
    mesh=plsc.VectorSubcoreMesh(core_axis_name="c", subcore_axis_name="s"),
    compiler_params=pltpu.CompilerParams(needs_layout_passes=False),
    scratch_types=[
        pltpu.VMEM((_HE,), jnp.int32),
        pltpu.VMEM((_HE,), jnp.int32),
        pltpu.VMEM((_ROWS_PER_W * N,), jnp.float32),
    ],
)(_count_sc_body)


def _ln(t, g, b, eps=1e-5):
    mu = jnp.mean(t, axis=-1, keepdims=True)
    d = t - mu
    var = jnp.mean(d * d, axis=-1, keepdims=True)
    return d * jax.lax.rsqrt(var + eps) * g + b


def _gelu(t):
    return 0.5 * t * (1.0 + jax.lax.erf(t * 0.7071067811865476))


def _fused_kernel(c_ref, x_ref,
                  w1a_ref, b1a_ref, g1a_ref, be1a_ref, w2a_ref, b2a_ref,
                  w1b_ref, b1b_ref, g1b_ref, be1b_ref, w2b_ref, b2b_ref,
                  ew1_ref, eb1_ref, g_ref, bes_ref, w2_ref, b2_ref,
                  o_ref,
                  ac_s, bct_s, acg_s, bcgt_s, va_s, vb_s, ben_s):
    i = pl.program_id(0)
    c = 0.7071067811865476

    @pl.when(i == 0)
    def _():
        C = c_ref[0] + c_ref[1]
        x = x_ref[...]

        def gin(h, w1, b1, g1, be1, w2, b2):
            t = jnp.dot(h, w1, preferred_element_type=jnp.float32) + b1
            t = _gelu(_ln(t, g1, be1))
            return jnp.dot(t, w2, preferred_element_type=jnp.float32) + b2

        agg1 = jnp.dot(C, x, preferred_element_type=jnp.float32)
        h = gin(x + agg1, w1a_ref[...], b1a_ref[...], g1a_ref[...],
                be1a_ref[...], w2a_ref[...], b2a_ref[...])
        agg2 = jnp.dot(C, h, preferred_element_type=jnp.float32)
        h = gin(h + agg2, w1b_ref[...], b1b_ref[...], g1b_ref[...],
                be1b_ref[...], w2b_ref[...], b2b_ref[...])
        ew1 = ew1_ref[...]
        g = g_ref[...]
        A = (jnp.dot(h, ew1[:DIM, :], preferred_element_type=jnp.float32)
             + eb1_ref[...])
        B = jnp.dot(h, ew1[DIM:, :], preferred_element_type=jnp.float32)
        Ac = A - jnp.mean(A, axis=-1, keepdims=True)
        va_s[...] = jnp.mean(Ac * Ac, axis=-1, keepdims=True)
        ac_s[...] = Ac
        acg_s[...] = Ac * g
        eye = jnp.eye(DIM, dtype=jnp.float32)
        BT = lax.dot_general(eye, B, (((1,), (1,)), ((), ())),
                             preferred_element_type=jnp.float32)
        ones_row = jnp.ones((1, DIM), jnp.float32)
        mB = lax.dot_general(ones_row, B, (((1,), (1,)), ((), ())),
                             preferred_element_type=jnp.float32) / DIM
        BcT = BT - mB
        bct_s[...] = BcT
        vb_s[...] = jnp.sum(BcT * BcT, axis=0, keepdims=True) / DIM
        bcgt_s[...] = BcT * g_ref[...].reshape(DIM, 1)
        ben_s[...] = jnp.broadcast_to(bes_ref[...], (DIM, N))

    ac = ac_s[pl.ds(i * BI, BI), :]
    acg = acg_s[pl.ds(i * BI, BI), :]
    va = va_s[pl.ds(i * BI, BI), :]
    bcgt = bcgt_s[...]
    P = lax.dot_general(ac, bct_s[...], (((1,), (0,)), ((), ())),
                        preferred_element_type=jnp.float32)
    var = va + vb_s[...] + (2.0 / DIM) * P
    rstd = jax.lax.rsqrt(var + 1e-5) * c
    s = ((acg[:, :, None] + bcgt[None, :, :]) * rstd[:, None, :]
         + ben_s[...][None, :, :])
    y = s + s * jax.lax.erf(s)
    w2 = w2_ref[...]
    rows = [lax.dot_general(w2, y[k], (((1,), (0,)), ((), ())),
                            preferred_element_type=jnp.float32)
            for k in range(BI)]
    o = jnp.concatenate(rows, axis=0)
    o_ref[...] = jax.nn.sigmoid(o + b2_ref[...])


def kernel(x, edge_index, c1_W1, c1_b1, c1_g1, c1_be1, c1_W2, c1_b2,
           c2_W1, c2_b1, c2_g1, c2_be1, c2_W2, c2_b2,
           e_W1, e_b1, e_g1, e_be1, e_W2, e_b2):
    ei = edge_index.astype(jnp.int32)
    src = ei[0]
    dst = ei[1]

    C = _count_sc(src, dst).reshape(_SC_CORES, N, N)

    r = lambda v: v.reshape(1, -1)
    full = lambda shape: pl.BlockSpec(shape, lambda i: tuple(0 for _ in shape))
    bes = (e_be1 * 0.7071067811865476).reshape(DIM, 1)
    out = pl.pallas_call(
        _fused_kernel,
        grid=(N // BI,),
        in_specs=[full((_SC_CORES, N, N)), full((N, DIN)),
                  full((DIN, DIM)), full((1, DIM)), full((1, DIM)),
                  full((1, DIM)), full((DIM, DIM)), full((1, DIM)),
                  full((DIM, DIM)), full((1, DIM)), full((1, DIM)),
                  full((1, DIM)), full((DIM, DIM)), full((1, DIM)),
                  full((2 * DIM, DIM)), full((1, DIM)), full((1, DIM)),
                  full((DIM, 1)), full((1, DIM)), full((1, 1))],
        out_specs=pl.BlockSpec((BI, N), lambda i: (i, 0)),
        out_shape=jax.ShapeDtypeStruct((N, N), jnp.float32),
        scratch_shapes=[pltpu.VMEM((N, DIM), jnp.float32),
                        pltpu.VMEM((DIM, N), jnp.float32),
                        pltpu.VMEM((N, DIM), jnp.float32),
                        pltpu.VMEM((DIM, N), jnp.float32),
                        pltpu.VMEM((N, 1), jnp.float32),
                        pltpu.VMEM((1, N), jnp.float32),
                        pltpu.VMEM((DIM, N), jnp.float32)],
    )(C, x,
      c1_W1, r(c1_b1), r(c1_g1), r(c1_be1), c1_W2, r(c1_b2),
      c2_W1, r(c2_b1), r(c2_g1), r(c2_be1), c2_W2, r(c2_b2),
      e_W1, r(e_b1), r(e_g1), bes,
      (e_W2 * 0.7071067811865476).reshape(1, DIM),
      e_b2.reshape(1, 1))
    return out


# speedup vs baseline: 2.8379x; 1.0188x over previous
import functools

import jax
import jax.numpy as jnp
from jax import lax
from jax.experimental import pallas as pl
from jax.experimental.pallas import tpu as pltpu
from jax.experimental.pallas import tpu_sc as plsc

N = 512
DIN = 512
DIM = 128
E = 16384
BI = 128

_SC_CORES = 2
_SC_SUBCORES = 16
_ROWS_PER_W = N // _SC_SUBCORES
_HE = E // _SC_CORES
_LANES = 16


def _count_sc_body(src_hbm, dst_hbm, out_hbm, src_v, dst_v, c_v):
    cid = lax.axis_index("c")
    sid = lax.axis_index("s")
    lo = sid * _ROWS_PER_W
    pltpu.sync_copy(src_hbm.at[pl.ds(cid * _HE, _HE)], src_v)
    pltpu.sync_copy(dst_hbm.at[pl.ds(cid * _HE, _HE)], dst_v)
    zeros = jnp.zeros((_LANES,), jnp.float32)

    @plsc.parallel_loop(0, _ROWS_PER_W * N // _LANES, unroll=8)
    def _zero(j):
        c_v[pl.ds(j * _LANES, _LANES)] = zeros

    @plsc.parallel_loop(0, _HE // _LANES, unroll=4)
    def _accum(i):
        d = dst_v[pl.ds(i * _LANES, _LANES)]
        s = src_v[pl.ds(i * _LANES, _LANES)]
        m = (d >= lo) & (d < lo + _ROWS_PER_W)
        idx = (d - lo) * N + s
        cnt, last = plsc.scan_count(idx, m)
        plsc.addupdate_scatter(c_v, [idx], cnt.astype(jnp.float32),
                               mask=last)

    pltpu.sync_copy(
        c_v, out_hbm.at[pl.ds((cid * N + lo) * N, _ROWS_PER_W * N)])


_count_sc = functools.partial(
    pl.kernel,
    out_type=jax.ShapeDtypeStruct((_SC_CORES * N * N,), jnp.float32),
    mesh=plsc.VectorSubcoreMesh(core_axis_name="c", subcore_axis_name="s"),
    compiler_params=pltpu.CompilerParams(needs_layout_passes=False),
    scratch_types=[
        pltpu.VMEM((_HE,), jnp.int32),
        pltpu.VMEM((_HE,), jnp.int32),
        pltpu.VMEM((_ROWS_PER_W * N,), jnp.float32),
    ],
)(_count_sc_body)


def _ln(t, g, b, eps=1e-5):
    mu = jnp.mean(t, axis=-1, keepdims=True)
    d = t - mu
    var = jnp.mean(d * d, axis=-1, keepdims=True)
    return d * jax.lax.rsqrt(var + eps) * g + b


def _gelu(t):
    return 0.5 * t * (1.0 + jax.lax.erf(t * 0.7071067811865476))


def _fused_kernel(c_ref, x_ref,
                  w1a_ref, b1a_ref, g1a_ref, be1a_ref, w2a_ref, b2a_ref,
                  w1b_ref, b1b_ref, g1b_ref, be1b_ref, w2b_ref, b2b_ref,
                  ew1_ref, eb1_ref, g_ref, bes_ref, w2_ref, b2_ref,
                  o_ref,
                  ac_s, bct_s, acg_s, bcgt_s, va_s, vb_s, ben_s):
    i = pl.program_id(0)
    c = 0.7071067811865476

    @pl.when(i == 0)
    def _():
        C = c_ref[0] + c_ref[1]
        x = x_ref[...]

        def gin(h, w1, b1, g1, be1, w2, b2):
            t = jnp.dot(h, w1, preferred_element_type=jnp.float32) + b1
            t = _gelu(_ln(t, g1, be1))
            return jnp.dot(t, w2, preferred_element_type=jnp.float32) + b2

        agg1 = jnp.dot(C, x, preferred_element_type=jnp.float32)
        h = gin(x + agg1, w1a_ref[...], b1a_ref[...], g1a_ref[...],
                be1a_ref[...], w2a_ref[...], b2a_ref[...])
        agg2 = jnp.dot(C, h, preferred_element_type=jnp.float32)
        h = gin(h + agg2, w1b_ref[...], b1b_ref[...], g1b_ref[...],
                be1b_ref[...], w2b_ref[...], b2b_ref[...])
        ew1 = ew1_ref[...]
        g = g_ref[...]
        A = (jnp.dot(h, ew1[:DIM, :], preferred_element_type=jnp.float32)
             + eb1_ref[...])
        B = jnp.dot(h, ew1[DIM:, :], preferred_element_type=jnp.float32)
        Ac = A - jnp.mean(A, axis=-1, keepdims=True)
        va_s[...] = jnp.mean(Ac * Ac, axis=-1, keepdims=True)
        ac_s[...] = Ac
        acg_s[...] = Ac * g
        eye = jnp.eye(DIM, dtype=jnp.float32)
        BT = lax.dot_general(eye, B, (((1,), (1,)), ((), ())),
                             preferred_element_type=jnp.float32)
        ones_row = jnp.ones((1, DIM), jnp.float32)
        mB = lax.dot_general(ones_row, B, (((1,), (1,)), ((), ())),
                             preferred_element_type=jnp.float32) / DIM
        BcT = BT - mB
        bct_s[...] = BcT
        vb_s[...] = jnp.sum(BcT * BcT, axis=0, keepdims=True) / DIM
        bcgt_s[...] = BcT * g_ref[...].reshape(DIM, 1)
        ben_s[...] = jnp.broadcast_to(bes_ref[...], (DIM, N))

    ac = ac_s[pl.ds(i * BI, BI), :]
    acg = acg_s[pl.ds(i * BI, BI), :]
    va = va_s[pl.ds(i * BI, BI), :]
    bcgt = bcgt_s[...]
    P = lax.dot_general(ac, bct_s[...], (((1,), (0,)), ((), ())),
                        preferred_element_type=jnp.float32)
    var = va + vb_s[...] + (2.0 / DIM) * P
    rstd = jax.lax.rsqrt(var + 1e-5) * c
    s = ((acg[:, :, None] + bcgt[None, :, :]) * rstd[:, None, :]
         + ben_s[...][None, :, :])
    y = s + s * jax.lax.erf(s)
    w2 = w2_ref[...]
    rows = [lax.dot_general(w2, y[k], (((1,), (0,)), ((), ())),
                            preferred_element_type=jnp.float32)
            for k in range(BI)]
    o = jnp.concatenate(rows, axis=0)
    o_ref[...] = jax.nn.sigmoid(o + b2_ref[...])


def kernel(x, edge_index, c1_W1, c1_b1, c1_g1, c1_be1, c1_W2, c1_b2,
           c2_W1, c2_b1, c2_g1, c2_be1, c2_W2, c2_b2,
           e_W1, e_b1, e_g1, e_be1, e_W2, e_b2):
    ei = edge_index.astype(jnp.int32)
    src = ei[0]
    dst = ei[1]

    C = _count_sc(src, dst).reshape(_SC_CORES, N, N)

    r = lambda v: v.reshape(1, -1)
    full = lambda shape: pl.BlockSpec(shape, lambda i: tuple(0 for _ in shape))
    bes = (e_be1 * 0.7071067811865476).reshape(DIM, 1)
    out = pl.pallas_call(
        _fused_kernel,
        grid=(N // BI,),
        in_specs=[full((_SC_CORES, N, N)), full((N, DIN)),
                  full((DIN, DIM)), full((1, DIM)), full((1, DIM)),
                  full((1, DIM)), full((DIM, DIM)), full((1, DIM)),
                  full((DIM, DIM)), full((1, DIM)), full((1, DIM)),
                  full((1, DIM)), full((DIM, DIM)), full((1, DIM)),
                  full((2 * DIM, DIM)), full((1, DIM)), full((1, DIM)),
                  full((DIM, 1)), full((1, DIM)), full((1, 1))],
        out_specs=pl.BlockSpec((BI, N), lambda i: (i, 0)),
        out_shape=jax.ShapeDtypeStruct((N, N), jnp.float32),
        scratch_shapes=[pltpu.VMEM((N, DIM), jnp.float32),
                        pltpu.VMEM((DIM, N), jnp.float32),
                        pltpu.VMEM((N, DIM), jnp.float32),
                        pltpu.VMEM((DIM, N), jnp.float32),
                        pltpu.VMEM((N, 1), jnp.float32),
                        pltpu.VMEM((1, N), jnp.float32),
                        pltpu.VMEM((DIM, N), jnp.float32)],
    )(C, x,
      c1_W1, r(c1_b1), r(c1_g1), r(c1_be1), c1_W2, r(c1_b2),
      c2_W1, r(c2_b1), r(c2_g1), r(c2_be1), c2_W2, r(c2_b2),
      e_W1, r(e_b1), r(e_g1), bes,
      (e_W2 * 0.7071067811865476).reshape(1, DIM),
      e_b2.reshape(1, 1))
    return out
